# baseline (speedup 1.0000x reference)
import functools

import jax
import jax.numpy as jnp
from jax import lax
from jax.experimental import pallas as pl
from jax.experimental.pallas import tpu as pltpu
from jax.experimental.pallas import tpu_sc as plsc

_TIMESTEPS = 1000
_B = 16384
_NC = 1
_NS = 16
_LANES = 16
_NW = _NC * _NS
_CHUNK = _B // _NW
_TAB = 1001
_MAGIC = 2.0 ** 23


@functools.partial(
    pl.kernel,
    mesh=plsc.VectorSubcoreMesh(
        core_axis_name="c", subcore_axis_name="s", num_cores=_NC
    ),
    out_type=jax.ShapeDtypeStruct((_B,), jnp.float32),
    compiler_params=pltpu.CompilerParams(needs_layout_passes=False),
    scratch_types=[
        pltpu.VMEM((_TAB,), jnp.float32),
        pltpu.VMEM((_CHUNK,), jnp.float32),
        pltpu.VMEM((_CHUNK,), jnp.float32),
        pltpu.SemaphoreType.DMA,
        pltpu.SemaphoreType.DMA,
    ],
)
def _lookup(t_hbm, gamma_hbm, out_hbm, gamma_v, t_v, out_v, sem_g, sem_t):
    base = lax.axis_index("s") * _CHUNK
    gcp = pltpu.async_copy(gamma_hbm, gamma_v, sem_g)
    tcp = pltpu.async_copy(t_hbm.at[pl.ds(base, _CHUNK)], t_v, sem_t)
    gcp.wait()
    tcp.wait()

    @plsc.parallel_loop(0, _CHUNK, _LANES, unroll=8)
    def _body(off):
        tv = t_v[pl.ds(off, _LANES)]
        x = tv * jnp.float32(_TIMESTEPS)
        r = (x + jnp.float32(_MAGIC)) - jnp.float32(_MAGIC)
        idx = r.astype(jnp.int32)
        idx = jnp.minimum(jnp.maximum(idx, 0), _TIMESTEPS)
        out_v[pl.ds(off, _LANES)] = plsc.load_gather(gamma_v, [idx])

    pltpu.sync_copy(out_v, out_hbm.at[pl.ds(base, _CHUNK)])


def kernel(t, gamma):
    out = _lookup(t.reshape(_B), gamma)
    return out.reshape(_B, 1)
